# baseline (speedup 1.0000x reference)
import jax
import jax.numpy as jnp
from jax import lax
from jax.experimental import pallas as pl
from jax.experimental.pallas import tpu as pltpu
from jax.experimental.pallas import tpu_sc as plsc

_NUM_CLASSES = 1000
_N_ROWS = 16384
_NC = 2
_NS = 16
_NW = _NC * _NS
_ROWS_PER_W = _N_ROWS // _NW
_WORDS_PER_W = _ROWS_PER_W * _NUM_CLASSES
_ZBLK_WORDS = _WORDS_PER_W // _NS
_L = 16
_IDX_ROW = 128
_N_IDX_ROWS = _ROWS_PER_W // _IDX_ROW


def _body(x_hbm, out_hbm, idx_v, pos_r, ones_v, zblk, zshared, sem_z, sem_o):
    cid = lax.axis_index("c")
    sid = lax.axis_index("s")
    wid = sid * _NC + cid
    base_row = wid * _ROWS_PER_W

    zvec = jnp.zeros((_L,), jnp.int32)
    onevec = jnp.full((_L,), 1, jnp.int32)
    lane1000 = lax.iota(jnp.int32, _L) * _NUM_CLASSES

    pltpu.sync_copy(x_hbm.at[pl.ds(base_row, _ROWS_PER_W)], idx_v)

    def _zero(i, carry):
        zblk[pl.ds(i * 2 * _L, _L)] = zvec
        zblk[pl.ds(i * 2 * _L + _L, _L)] = zvec
        return carry

    lax.fori_loop(0, _ZBLK_WORDS // (2 * _L), _zero, 0)

    for k in range(_IDX_ROW // _L):
        ones_v[pl.ds(k * _L, _L)] = onevec
    base_word = base_row * _NUM_CLASSES
    for k in range(_ROWS_PER_W // _L):
        xv = idx_v[pl.ds(k * _L, _L)]
        pos = base_word + k * _L * _NUM_CLASSES + lane1000 + xv
        pos_r[k * _L // _IDX_ROW, pl.ds((k * _L) % _IDX_ROW, _L)] = pos

    pltpu.sync_copy(zblk, zshared.at[pl.ds(sid * _ZBLK_WORDS, _ZBLK_WORDS)])
    plsc.subcore_barrier()

    zdma = pltpu.make_async_copy(
        zshared, out_hbm.at[pl.ds(base_word, _WORDS_PER_W)], sem_z)
    zdma.start()
    zdma.wait()

    odmas = []
    for j in range(_N_IDX_ROWS):
        d = pltpu.make_async_copy(ones_v, out_hbm.at[pos_r.at[j]], sem_o)
        d.start()
        odmas.append(d)
    for d in odmas:
        d.wait()


@jax.jit
def kernel(x):
    mesh = plsc.VectorSubcoreMesh(
        core_axis_name="c", subcore_axis_name="s",
        num_cores=_NC, num_subcores=_NS)
    flat = pl.kernel(
        _body,
        out_type=jax.ShapeDtypeStruct((_N_ROWS * _NUM_CLASSES,), jnp.int32),
        mesh=mesh,
        scratch_types=[
            pltpu.VMEM((_ROWS_PER_W,), jnp.int32),
            pltpu.VMEM((_N_IDX_ROWS, _IDX_ROW), jnp.int32),
            pltpu.VMEM((_IDX_ROW,), jnp.int32),
            pltpu.VMEM((_ZBLK_WORDS,), jnp.int32),
            pltpu.VMEM_SHARED((_WORDS_PER_W,), jnp.int32),
            pltpu.SemaphoreType.DMA,
            pltpu.SemaphoreType.DMA,
        ],
        compiler_params=pltpu.CompilerParams(needs_layout_passes=False),
    )(x)
    return flat.reshape(_N_ROWS, _NUM_CLASSES)


# speedup vs baseline: 1.7932x; 1.7932x over previous
import jax
import jax.numpy as jnp
from jax import lax
from jax.experimental import pallas as pl
from jax.experimental.pallas import tpu as pltpu
from jax.experimental.pallas import tpu_sc as plsc

_NUM_CLASSES = 1000
_N_ROWS = 16384
_NC = 2
_NS = 16
_NW = _NC * _NS
_ROWS_PER_W = _N_ROWS // _NW
_CHUNK_ROWS = 32
_N_CHUNKS = _ROWS_PER_W // _CHUNK_ROWS
_L = 16


def _body(x_hbm, out_hbm, idx_v, buf0, buf1, sem0, sem1):
    wid = lax.axis_index("s") * _NC + lax.axis_index("c")
    base_row = wid * _ROWS_PER_W

    zvec = jnp.zeros((_L,), jnp.int32)
    onevec = jnp.full((_L,), 1, jnp.int32)
    lane = lax.iota(jnp.int32, _L)

    pltpu.sync_copy(x_hbm.at[pl.ds(base_row, _ROWS_PER_W)], idx_v)

    def _zero(r, carry):
        for k in range(_NUM_CLASSES // _L):
            buf0[r, pl.ds(k * _L, _L)] = zvec
            buf1[r, pl.ds(k * _L, _L)] = zvec
        buf0[r, pl.ds(_NUM_CLASSES - _L, _L)] = zvec
        buf1[r, pl.ds(_NUM_CLASSES - _L, _L)] = zvec
        return carry

    lax.fori_loop(0, _CHUNK_ROWS, _zero, 0)

    bufs = (buf0, buf1)
    sems = (sem0, sem1)
    dmas = {}
    prev_cols = {}
    for c in range(_N_CHUNKS):
        b = c % 2
        buf = bufs[b]
        if c >= 2:
            dmas[b].wait()
            for v, xv_old in enumerate(prev_cols[b]):
                plsc.store_scatter(buf, [lane + v * _L, xv_old], zvec)
        cols = []
        for v in range(_CHUNK_ROWS // _L):
            xv = idx_v[pl.ds(c * _CHUNK_ROWS + v * _L, _L)]
            plsc.store_scatter(buf, [lane + v * _L, xv], onevec)
            cols.append(xv)
        prev_cols[b] = cols
        row0 = base_row + c * _CHUNK_ROWS
        dma = pltpu.make_async_copy(
            buf, out_hbm.at[pl.ds(row0, _CHUNK_ROWS), :], sems[b])
        dma.start()
        dmas[b] = dma
    dmas[0].wait()
    dmas[1].wait()


@jax.jit
def kernel(x):
    mesh = plsc.VectorSubcoreMesh(
        core_axis_name="c", subcore_axis_name="s",
        num_cores=_NC, num_subcores=_NS)
    return pl.kernel(
        _body,
        out_type=jax.ShapeDtypeStruct((_N_ROWS, _NUM_CLASSES), jnp.int32),
        mesh=mesh,
        scratch_types=[
            pltpu.VMEM((_ROWS_PER_W,), jnp.int32),
            pltpu.VMEM((_CHUNK_ROWS, _NUM_CLASSES), jnp.int32),
            pltpu.VMEM((_CHUNK_ROWS, _NUM_CLASSES), jnp.int32),
            pltpu.SemaphoreType.DMA,
            pltpu.SemaphoreType.DMA,
        ],
        compiler_params=pltpu.CompilerParams(needs_layout_passes=False),
    )(x)
